# Initial kernel scaffold; baseline (speedup 1.0000x reference)
import jax
import jax.numpy as jnp
from jax.experimental import pallas as pl


def kernel(x, params, edge_index, regression_mask):
    raise NotImplementedError("write your pallas kernel here")


# speedup vs baseline: 6.1398x; 6.1398x over previous
import functools

import jax
import jax.numpy as jnp
from jax import lax
from jax.experimental import pallas as pl
from jax.experimental.pallas import tpu as pltpu
from jax.experimental.pallas import tpu_sc as plsc

N = 10000
D = 256
E = 160000
NC = 2
NT = 16
K = 128
NB = 80
E_PAD = NT * NB * K
GP = 8
NG = NB // GP
CW = 128
NCH = (2 * D) // CW
RPT = 632
R_ACC = NT * RPT
BLK = 2000
GRID = N // BLK
EPS = 1e-7


def _sc_body(tab, srcb, dstb, zer, out, src_v, dst_g, rows_v, acc,
             g0, g1, d0, d1):
    gsem = [g0, g1]
    dsem = [d0, d1]
    core = lax.axis_index("c")
    sub = lax.axis_index("s")
    dst_sub = dstb.at[sub]

    pltpu.sync_copy(srcb.at[sub], src_v)

    def dst_load(g, buf):
        return pltpu.async_copy(dst_sub.at[pl.ds(g * GP, GP)],
                                dst_g.at[buf], dsem[buf])

    def dst_wait(g, buf):
        pltpu.make_async_copy(dst_sub.at[pl.ds(g * GP, GP)],
                              dst_g.at[buf], dsem[buf]).wait()

    for p in range(NCH // NC):
        pltpu.sync_copy(zer, acc.at[pl.ds(sub * RPT, RPT)])
        plsc.subcore_barrier()

        for cl in range(NC):
            chunk = NC * p + cl

            @pl.when(core == cl)
            def _(chunk=chunk):
                tab_c = tab.at[chunk]

                def gath(b, slot):
                    return pltpu.async_copy(tab_c.at[src_v.at[b]],
                                            rows_v.at[slot], gsem[slot])

                def gath_wait(b, slot):
                    pltpu.make_async_copy(tab_c.at[src_v.at[b]],
                                          rows_v.at[slot], gsem[slot]).wait()

                dst_load(0, 0)
                gath(0, 0)
                gath(1, 1)

                @pl.loop(0, NG // 2)
                def _(s):
                    for half in range(2):
                        g = 2 * s + half
                        buf = half
                        dst_wait(g, buf)

                        @pl.when(g + 1 < NG)
                        def _():
                            dst_load(g + 1, 1 - buf)

                        for r in range(GP):
                            b = g * GP + r
                            slot = r % 2
                            gath_wait(b, slot)
                            pltpu.sync_copy(rows_v.at[slot],
                                            acc.at[dst_g.at[buf].at[r]],
                                            add=True)

                            @pl.when(b + 2 < NB)
                            def _():
                                gath(b + 2, slot)

        plsc.subcore_barrier()

        for cl in range(NC):
            chunk = 2 * p + cl

            @pl.when(core == cl)
            def _(chunk=chunk):
                sl = pl.ds(sub * RPT, RPT)
                pltpu.sync_copy(acc.at[sl], out.at[chunk].at[sl])

        plsc.subcore_barrier()


def _sc_aggregate(tables, srcr, dstr, zeros):
    mesh = plsc.VectorSubcoreMesh(core_axis_name="c", subcore_axis_name="s",
                                  num_cores=NC, num_subcores=NT)
    call = pl.kernel(
        _sc_body,
        out_type=jax.ShapeDtypeStruct((NCH, R_ACC, CW), jnp.float32),
        mesh=mesh,
        scratch_types=[
            pltpu.VMEM((NB, K), jnp.int32),
            pltpu.VMEM((2, GP, K), jnp.int32),
            pltpu.VMEM((2, K, CW), jnp.float32),
            pltpu.VMEM_SHARED((R_ACC, CW), jnp.float32),
            pltpu.SemaphoreType.DMA,
            pltpu.SemaphoreType.DMA,
            pltpu.SemaphoreType.DMA,
            pltpu.SemaphoreType.DMA,
        ],
    )
    return call(tables, srcr, dstr, zeros)


def _ln(h, g, b):
    mu = jnp.mean(h, axis=-1, keepdims=True)
    var = jnp.mean((h - mu) ** 2, axis=-1, keepdims=True)
    return (h - mu) / jnp.sqrt(var + 1e-5) * g + b


def _colmax_body(g_ref, t_ref, o_ref):
    s = (jnp.maximum(g_ref[...], 0.0) + EPS) * t_ref[0]
    m = jnp.broadcast_to(jnp.max(s, axis=0, keepdims=True), (8, D))

    @pl.when(pl.program_id(0) == 0)
    def _():
        o_ref[...] = m

    @pl.when(pl.program_id(0) != 0)
    def _():
        o_ref[...] = jnp.maximum(o_ref[...], m)


def _colmax(g, t):
    return pl.pallas_call(
        _colmax_body,
        grid=(GRID,),
        in_specs=[
            pl.BlockSpec((BLK, D), lambda i: (i, 0)),
            pl.BlockSpec(memory_space=pltpu.SMEM),
        ],
        out_specs=pl.BlockSpec((8, D), lambda i: (0, 0)),
        out_shape=jax.ShapeDtypeStruct((8, D), jnp.float32),
    )(g, t)


def _prep_body(g_ref, cm_ref, t_ref, tab_ref):
    T = jnp.maximum(g_ref[...], 0.0) + EPS
    cm = cm_ref[0, :]
    A = jnp.exp(T * t_ref[0] - cm[None, :])
    B = A * T
    for c in range(NCH // 2):
        tab_ref[c, :, :] = A[:, c * CW:(c + 1) * CW]
        tab_ref[NCH // 2 + c, :, :] = B[:, c * CW:(c + 1) * CW]


def _prep_tables(g, cmax, t):
    return pl.pallas_call(
        _prep_body,
        grid=(GRID,),
        in_specs=[
            pl.BlockSpec((BLK, D), lambda i: (i, 0)),
            pl.BlockSpec((8, D), lambda i: (0, 0)),
            pl.BlockSpec(memory_space=pltpu.SMEM),
        ],
        out_specs=pl.BlockSpec((NCH, BLK, CW), lambda i: (0, i, 0)),
        out_shape=jax.ShapeDtypeStruct((NCH, N, CW), jnp.float32),
    )(g, cmax, t)


def _aggr_from_acc(acc_ref):
    h = NCH // 2
    den = jnp.concatenate([acc_ref[c] for c in range(h)], axis=1)
    num = jnp.concatenate([acc_ref[h + c] for c in range(h)], axis=1)
    return num / (den + 1e-16)


def _conv_mlp(aggr, g, W1, b1, l1g, l1b, W2, b2):
    out = aggr + g
    h = jnp.dot(out, W1, preferred_element_type=jnp.float32) + b1
    h = _ln(h, l1g, l1b)
    h = jnp.maximum(h, 0.0)
    return jnp.dot(h, W2, preferred_element_type=jnp.float32) + b2


def _k3_body(acc_ref, g_ref, xp_ref, W1, b1, l1g, l1b, W2, b2,
             png, pnb, nng, nnb, t_ref, x_ref, g2_ref, cm_ref, *, first):
    aggr = _aggr_from_acc(acc_ref)
    conv = _conv_mlp(aggr, g_ref[...], W1[...], b1[...], l1g[...], l1b[...],
                     W2[...], b2[...])
    if first:
        x_new = jnp.maximum(_ln(conv, png[...], pnb[...]), 0.0)
    else:
        x_new = xp_ref[...] + conv
    g_next = jnp.maximum(_ln(x_new, nng[...], nnb[...]), 0.0)
    x_ref[...] = x_new
    g2_ref[...] = g_next

    s = (jnp.maximum(g_next, 0.0) + EPS) * t_ref[0]
    m = jnp.broadcast_to(jnp.max(s, axis=0, keepdims=True), (8, D))

    @pl.when(pl.program_id(0) == 0)
    def _():
        cm_ref[...] = m

    @pl.when(pl.program_id(0) != 0)
    def _():
        cm_ref[...] = jnp.maximum(cm_ref[...], m)


def _k3_layer(acc, g, xprev, p, nxt_g, nxt_b, t_next, first):
    row = lambda i: (i, 0)
    return pl.pallas_call(
        functools.partial(_k3_body, first=first),
        grid=(GRID,),
        in_specs=[
            pl.BlockSpec((NCH, BLK, CW), lambda i: (0, i, 0)),
            pl.BlockSpec((BLK, D), row),
            pl.BlockSpec((BLK, D), row),
            pl.BlockSpec((D, 2 * D), lambda i: (0, 0)),
            pl.BlockSpec((1, 2 * D), lambda i: (0, 0)),
            pl.BlockSpec((1, 2 * D), lambda i: (0, 0)),
            pl.BlockSpec((1, 2 * D), lambda i: (0, 0)),
            pl.BlockSpec((2 * D, D), lambda i: (0, 0)),
            pl.BlockSpec((1, D), lambda i: (0, 0)),
            pl.BlockSpec((1, D), lambda i: (0, 0)),
            pl.BlockSpec((1, D), lambda i: (0, 0)),
            pl.BlockSpec((1, D), lambda i: (0, 0)),
            pl.BlockSpec((1, D), lambda i: (0, 0)),
            pl.BlockSpec(memory_space=pltpu.SMEM),
        ],
        out_specs=[
            pl.BlockSpec((BLK, D), row),
            pl.BlockSpec((BLK, D), row),
            pl.BlockSpec((8, D), lambda i: (0, 0)),
        ],
        out_shape=[
            jax.ShapeDtypeStruct((N, D), jnp.float32),
            jax.ShapeDtypeStruct((N, D), jnp.float32),
            jax.ShapeDtypeStruct((8, D), jnp.float32),
        ],
    )(acc, g, xprev, p['W1'], p['b1'], p['ln1_g'], p['ln1_b'], p['W2'],
      p['b2'], p['norm_g'], p['norm_b'], nxt_g, nxt_b, t_next)


def _k3f_body(acc_ref, g_ref, xp_ref, W1, b1, l1g, l1b, W2, b2,
              lW1, lb1, lg1, lbe1, lW2, lb2, lg2, lbe2, hW, hb_ref, m_ref,
              o_ref):
    aggr = _aggr_from_acc(acc_ref)
    conv = _conv_mlp(aggr, g_ref[...], W1[...], b1[...], l1g[...], l1b[...],
                     W2[...], b2[...])
    x = xp_ref[...] + conv
    x = jnp.maximum(
        _ln(jnp.dot(x, lW1[...], preferred_element_type=jnp.float32)
            + lb1[...], lg1[...], lbe1[...]), 0.0)
    x = jnp.maximum(
        _ln(jnp.dot(x, lW2[...], preferred_element_type=jnp.float32)
            + lb2[...], lg2[...], lbe2[...]), 0.0)
    v = jnp.sum(x * hW[...], axis=1) + hb_ref[0]
    o_ref[...] = jnp.where(m_ref[...] > 0.0, v[:, None], 0.0)


def _k3_final(acc, g, xprev, p, lin, head_w_row, head_b, mask2d):
    row = lambda i: (i, 0)
    cst = lambda i: (0, 0)
    l1, l2 = lin
    return pl.pallas_call(
        _k3f_body,
        grid=(GRID,),
        in_specs=[
            pl.BlockSpec((NCH, BLK, CW), lambda i: (0, i, 0)),
            pl.BlockSpec((BLK, D), row),
            pl.BlockSpec((BLK, D), row),
            pl.BlockSpec((D, 2 * D), cst),
            pl.BlockSpec((1, 2 * D), cst),
            pl.BlockSpec((1, 2 * D), cst),
            pl.BlockSpec((1, 2 * D), cst),
            pl.BlockSpec((2 * D, D), cst),
            pl.BlockSpec((1, D), cst),
            pl.BlockSpec((D, D), cst),
            pl.BlockSpec((1, D), cst),
            pl.BlockSpec((1, D), cst),
            pl.BlockSpec((1, D), cst),
            pl.BlockSpec((D, D), cst),
            pl.BlockSpec((1, D), cst),
            pl.BlockSpec((1, D), cst),
            pl.BlockSpec((1, D), cst),
            pl.BlockSpec((1, D), cst),
            pl.BlockSpec(memory_space=pltpu.SMEM),
            pl.BlockSpec((BLK, 1), lambda i: (i, 0)),
        ],
        out_specs=pl.BlockSpec((BLK, 1), lambda i: (i, 0)),
        out_shape=jax.ShapeDtypeStruct((N, 1), jnp.float32),
    )(acc, g, xprev, p['W1'], p['b1'], p['ln1_g'], p['ln1_b'], p['W2'],
      p['b2'], l1['W'], l1['b'], l1['g'], l1['be'], l2['W'], l2['b'],
      l2['g'], l2['be'], head_w_row, head_b, mask2d)


def kernel(x, params, edge_index, regression_mask):
    convs = params['convs']
    lin = params['lin']

    def v2(a):
        return jnp.reshape(a, (1, -1))

    cp = []
    for p in convs:
        cp.append({
            'W1': p['W1'], 'W2': p['W2'],
            'b1': v2(p['b1']), 'b2': v2(p['b2']),
            'ln1_g': v2(p['ln1_g']), 'ln1_b': v2(p['ln1_b']),
            'norm_g': v2(p['norm_g']), 'norm_b': v2(p['norm_b']),
            't': jnp.reshape(p['t'], (1,)),
        })
    lp = [{'W': q['W'], 'b': v2(q['b']), 'g': v2(q['g']), 'be': v2(q['be'])}
          for q in lin]
    head_w_row = jnp.reshape(params['head_W'], (1, D))
    head_b = jnp.reshape(params['head_b'], (1,))

    src = edge_index[0]
    dst = edge_index[1]
    pad = E_PAD - E
    srcr = jnp.concatenate([src, jnp.zeros((pad,), jnp.int32)]
                           ).reshape(NT, NB, K)
    dstr = jnp.concatenate([dst, jnp.full((pad,), N, jnp.int32)]
                           ).reshape(NT, NB, K)
    zeros = jnp.zeros((RPT, CW), jnp.float32)
    mask2d = regression_mask.astype(jnp.float32)[:, None]

    cmax = _colmax(x, cp[0]['t'])
    tables = _prep_tables(x, cmax, cp[0]['t'])
    acc = _sc_aggregate(tables, srcr, dstr, zeros)
    x1, g2, cmax2 = _k3_layer(acc, x, x, cp[0], cp[1]['norm_g'],
                              cp[1]['norm_b'], cp[1]['t'], first=True)

    tables = _prep_tables(g2, cmax2, cp[1]['t'])
    acc = _sc_aggregate(tables, srcr, dstr, zeros)
    x2, g3, cmax3 = _k3_layer(acc, g2, x1, cp[1], cp[2]['norm_g'],
                              cp[2]['norm_b'], cp[2]['t'], first=False)

    tables = _prep_tables(g3, cmax3, cp[2]['t'])
    acc = _sc_aggregate(tables, srcr, dstr, zeros)
    out2d = _k3_final(acc, g3, x2, cp[2], lp, head_w_row, head_b, mask2d)
    return out2d[:, 0]
